# baseline (speedup 1.0000x reference)
import functools

import jax
import jax.numpy as jnp
from jax import lax
from jax.experimental import pallas as pl
from jax.experimental.pallas import tpu as pltpu
from jax.experimental.pallas import tpu_sc as plsc

_AF = 128
_NF = 16
_M = 32
_N = 10000
_NPAD = 10240
_EPAD = _NPAD * _M
_EDGES = _N * _M
_HID = 256
_NCONV = 3
_BATCH = 32
_L = 256
_MAXG = 512
_BLK = 128
_GRID = _NPAD // _BLK
_BLKR = 512
_GRIDR = _NPAD // _BLKR

_NC, _NS = 2, 16
_NW = _NC * _NS


def _make_sc_gather(n_rows, table_shape, dtype=jnp.float32):
    b_per_w = n_rows // _NW
    chunk = 128
    while b_per_w % chunk != 0:
        chunk -= 8
    n_chunks = b_per_w // chunk
    assert n_chunks % 2 == 0
    d = table_shape[1]
    mesh = plsc.VectorSubcoreMesh(core_axis_name="c", subcore_axis_name="s")

    @functools.partial(
        pl.kernel,
        mesh=mesh,
        compiler_params=pltpu.CompilerParams(use_tc_tiling_on_sc=False),
        out_type=jax.ShapeDtypeStruct((n_rows, d), dtype),
        scratch_types=[
            pltpu.VMEM((b_per_w,), jnp.int32),
            pltpu.VMEM((chunk, d), dtype),
            pltpu.VMEM((chunk, d), dtype),
            pltpu.SemaphoreType.DMA,
            pltpu.SemaphoreType.DMA,
            pltpu.SemaphoreType.DMA,
            pltpu.SemaphoreType.DMA,
        ],
    )
    def gather_k(table_hbm, idx_hbm, out_hbm, idx_all, rows0, rows1,
                 sg0, sg1, so0, so1):
        wid = lax.axis_index("s") * _NC + lax.axis_index("c")
        base_w = wid * b_per_w
        pltpu.sync_copy(idx_hbm.at[pl.ds(base_w, b_per_w)], idx_all)
        rows = (rows0, rows1)
        sg = (sg0, sg1)
        so = (so0, so1)

        def gather_cp(c, b):
            return pltpu.make_async_copy(
                table_hbm.at[idx_all.at[pl.ds(c * chunk, chunk)]],
                rows[b], sg[b])

        gather_cp(0, 0).start()
        gather_cp(1, 1).start()

        def slot(c, b):
            gather_cp(c, b).wait()
            pltpu.async_copy(
                rows[b], out_hbm.at[pl.ds(base_w + c * chunk, chunk)],
                so[b]).wait()

            @pl.when(c + 2 < n_chunks)
            def _():
                gather_cp(c + 2, b).start()

        def body(t, carry):
            slot(t * 2, 0)
            slot(t * 2 + 1, 1)
            return carry

        lax.fori_loop(0, n_chunks // 2, body, 0)

    return gather_k


def _softplus(x):
    return jnp.maximum(x, 0.0) + jnp.log1p(jnp.exp(-jnp.abs(x)))


def _pack_bf16(x):
    u = lax.bitcast_convert_type(x, jnp.uint32)
    u = u + jnp.uint32(0x7FFF) + ((u >> 16) & jnp.uint32(1))
    lo = u[:, :_AF // 2] >> 16
    hi = u[:, _AF // 2:] & jnp.uint32(0xFFFF0000)
    return lax.bitcast_convert_type(lo | hi, jnp.float32)


def _unpack_matmul2(gp, wn):
    u = lax.bitcast_convert_type(gp, jnp.uint32)
    lo = lax.bitcast_convert_type(u << 16, jnp.float32).astype(jnp.bfloat16)
    hi = lax.bitcast_convert_type(u & jnp.uint32(0xFFFF0000),
                                  jnp.float32).astype(jnp.bfloat16)
    h = _AF // 2
    z_e = (jnp.dot(lo[:, :h], wn[:h], preferred_element_type=jnp.float32)
           + jnp.dot(hi[:, :h], wn[h:], preferred_element_type=jnp.float32))
    z_o = (jnp.dot(lo[:, h:], wn[:h], preferred_element_type=jnp.float32)
           + jnp.dot(hi[:, h:], wn[h:], preferred_element_type=jnp.float32))
    return z_e, z_o


def _rowmask(i, blk):
    aid = i * blk + lax.broadcasted_iota(jnp.int32, (blk, 1), 0)
    return (aid < _N).astype(jnp.float32)


def _k0_body(x_ref, w_ref, b_ref, o_ref, xb_ref):
    x = x_ref[...] * _rowmask(pl.program_id(0), _BLKR)
    o_ref[...] = (
        jnp.dot(x, w_ref[...], preferred_element_type=jnp.float32)
        + b_ref[...]
    ) * _rowmask(pl.program_id(0), _BLKR)
    xb_ref[...] = _pack_bf16(x)


def _matmul_bias(x, w, b):
    kdim, odim = w.shape
    return pl.pallas_call(
        _k0_body,
        grid=(_GRIDR,),
        in_specs=[
            pl.BlockSpec((_BLKR, kdim), lambda i: (i, 0)),
            pl.BlockSpec((kdim, odim), lambda i: (0, 0)),
            pl.BlockSpec((1, odim), lambda i: (0, 0)),
        ],
        out_specs=[
            pl.BlockSpec((_BLKR, odim), lambda i: (i, 0)),
            pl.BlockSpec((_BLKR, kdim // 2), lambda i: (i, 0)),
        ],
        out_shape=[
            jax.ShapeDtypeStruct((_NPAD, odim), jnp.float32),
            jax.ShapeDtypeStruct((_NPAD, kdim // 2), jnp.float32),
        ],
    )(x, w, b)


def _stats_body(g_ref, xfe_ref, xfo_ref, p_ref, wn_ref, wnf_ref,
                s1_ref, s2_ref):
    i = pl.program_id(0)

    @pl.when(i == 0)
    def _():
        s1_ref[...] = jnp.zeros_like(s1_ref)
        s2_ref[...] = jnp.zeros_like(s2_ref)

    z_e, z_o = _unpack_matmul2(g_ref[...], wn_ref[...])
    e_e = lax.dot_general(xfe_ref[...], wnf_ref[...], (((0,), (0,)), ((), ())),
                          preferred_element_type=jnp.float32)
    e_o = lax.dot_general(xfo_ref[...], wnf_ref[...], (((0,), (0,)), ((), ())),
                          preferred_element_type=jnp.float32)
    pb = p_ref[...][:, None, :]
    ge = (z_e + e_e).reshape(_BLK, _M // 2, 2 * _AF) + pb
    go = (z_o + e_o).reshape(_BLK, _M // 2, 2 * _AF) + pb
    s1_ref[...] += (jnp.sum(ge, axis=(0, 1))
                    + jnp.sum(go, axis=(0, 1)))[None, :]
    s2_ref[...] += (jnp.sum(ge * ge, axis=(0, 1))
                    + jnp.sum(go * go, axis=(0, 1)))[None, :]


def _edge_stats(gat, xfe, xfo, p, wn, wnf):
    return pl.pallas_call(
        _stats_body,
        grid=(_GRID,),
        in_specs=[
            pl.BlockSpec((_BLK * _M // 2, _AF), lambda i: (i, 0)),
            pl.BlockSpec((_NF, _BLK * _M // 2), lambda i: (0, i)),
            pl.BlockSpec((_NF, _BLK * _M // 2), lambda i: (0, i)),
            pl.BlockSpec((_BLK, 2 * _AF), lambda i: (i, 0)),
            pl.BlockSpec((_AF, 2 * _AF), lambda i: (0, 0)),
            pl.BlockSpec((_NF, 2 * _AF), lambda i: (0, 0)),
        ],
        out_specs=[
            pl.BlockSpec((1, 2 * _AF), lambda i: (0, 0)),
            pl.BlockSpec((1, 2 * _AF), lambda i: (0, 0)),
        ],
        out_shape=[
            jax.ShapeDtypeStruct((1, 2 * _AF), jnp.float32),
            jax.ShapeDtypeStruct((1, 2 * _AF), jnp.float32),
        ],
    )(gat, xfe, xfo, p, wn, wnf)


def _apply_body(g_ref, xfe_ref, xfo_ref, p_ref, wn_ref, wnf_ref, ac_ref,
                ns_ref, t1_ref, t2_ref):
    i = pl.program_id(0)

    @pl.when(i == 0)
    def _():
        t1_ref[...] = jnp.zeros_like(t1_ref)
        t2_ref[...] = jnp.zeros_like(t2_ref)

    z_e, z_o = _unpack_matmul2(g_ref[...], wn_ref[...])
    e_e = lax.dot_general(xfe_ref[...], wnf_ref[...], (((0,), (0,)), ((), ())),
                          preferred_element_type=jnp.float32)
    e_o = lax.dot_general(xfo_ref[...], wnf_ref[...], (((0,), (0,)), ((), ())),
                          preferred_element_type=jnp.float32)
    pb = (p_ref[...] * ac_ref[0:1, :] + ac_ref[1:2, :])[:, None, :]
    ghe = (z_e + e_e).reshape(_BLK, _M // 2, 2 * _AF) + pb
    gho = (z_o + e_o).reshape(_BLK, _M // 2, 2 * _AF) + pb
    red = (jnp.sum(jax.nn.sigmoid(ghe[:, :, :_AF])
                   * _softplus(ghe[:, :, _AF:]), axis=1)
           + jnp.sum(jax.nn.sigmoid(gho[:, :, :_AF])
                     * _softplus(gho[:, :, _AF:]), axis=1))
    ns_ref[...] = red
    aid = i * _BLK + lax.broadcasted_iota(jnp.int32, (_BLK, 1), 0)
    msk = (aid < _N).astype(jnp.float32)
    rm = red * msk
    t1_ref[...] += jnp.sum(rm, axis=0)[None, :]
    t2_ref[...] += jnp.sum(rm * red, axis=0)[None, :]


def _edge_apply(gat, xfe, xfo, p, wn, wnf, ac):
    return pl.pallas_call(
        _apply_body,
        grid=(_GRID,),
        in_specs=[
            pl.BlockSpec((_BLK * _M // 2, _AF), lambda i: (i, 0)),
            pl.BlockSpec((_NF, _BLK * _M // 2), lambda i: (0, i)),
            pl.BlockSpec((_NF, _BLK * _M // 2), lambda i: (0, i)),
            pl.BlockSpec((_BLK, 2 * _AF), lambda i: (i, 0)),
            pl.BlockSpec((_AF, 2 * _AF), lambda i: (0, 0)),
            pl.BlockSpec((_NF, 2 * _AF), lambda i: (0, 0)),
            pl.BlockSpec((2, 2 * _AF), lambda i: (0, 0)),
        ],
        out_specs=[
            pl.BlockSpec((_BLK, _AF), lambda i: (i, 0)),
            pl.BlockSpec((1, _AF), lambda i: (0, 0)),
            pl.BlockSpec((1, _AF), lambda i: (0, 0)),
        ],
        out_shape=[
            jax.ShapeDtypeStruct((_NPAD, _AF), jnp.float32),
            jax.ShapeDtypeStruct((1, _AF), jnp.float32),
            jax.ShapeDtypeStruct((1, _AF), jnp.float32),
        ],
    )(gat, xfe, xfo, p, wn, wnf, ac)


def _update_body(af_ref, ns_ref, ac2_ref, w_ref, b_ref,
                 afn_ref, pn_ref, afb_ref):
    a2 = ac2_ref[0:1, :]
    c2 = ac2_ref[1:2, :]
    m = _rowmask(pl.program_id(0), _BLKR)
    afn = _softplus(af_ref[...] + ns_ref[...] * a2 + c2) * m
    afn_ref[...] = afn
    afb_ref[...] = _pack_bf16(afn)
    pn_ref[...] = (
        jnp.dot(afn, w_ref[...], preferred_element_type=jnp.float32)
        + b_ref[...]
    ) * m


def _update(af, ns, ac2, w, b):
    odim = w.shape[1]
    return pl.pallas_call(
        _update_body,
        grid=(_GRIDR,),
        in_specs=[
            pl.BlockSpec((_BLKR, _AF), lambda i: (i, 0)),
            pl.BlockSpec((_BLKR, _AF), lambda i: (i, 0)),
            pl.BlockSpec((2, _AF), lambda i: (0, 0)),
            pl.BlockSpec((_AF, odim), lambda i: (0, 0)),
            pl.BlockSpec((1, odim), lambda i: (0, 0)),
        ],
        out_specs=[
            pl.BlockSpec((_BLKR, _AF), lambda i: (i, 0)),
            pl.BlockSpec((_BLKR, odim), lambda i: (i, 0)),
            pl.BlockSpec((_BLKR, _AF // 2), lambda i: (i, 0)),
        ],
        out_shape=[
            jax.ShapeDtypeStruct((_NPAD, _AF), jnp.float32),
            jax.ShapeDtypeStruct((_NPAD, odim), jnp.float32),
            jax.ShapeDtypeStruct((_NPAD, _AF // 2), jnp.float32),
        ],
    )(af, ns, ac2, w, b)


def kernel(atom_num, nbr_idx, nbr_fea, crystal_atom_idx, embedding,
           convW, convB, bn1_g, bn1_b, bn2_g, bn2_b, fcW, fcb):
    f32 = jnp.float32
    an_pad = jnp.concatenate(
        [atom_num.astype(jnp.int32), jnp.zeros((_NPAD - _N,), jnp.int32)])
    flat_idx = jnp.concatenate(
        [nbr_idx.reshape(-1).astype(jnp.int32),
         jnp.full((_EPAD - _EDGES,), _NPAD - 1, jnp.int32)])
    x2 = nbr_fea.reshape(_EDGES, _NF)
    xfe = jnp.concatenate(
        [x2[0::2].T, jnp.zeros((_NF, (_EPAD - _EDGES) // 2), f32)], axis=1)
    xfo = jnp.concatenate(
        [x2[1::2].T, jnp.zeros((_NF, (_EPAD - _EDGES) // 2), f32)], axis=1)

    embed_gather = _make_sc_gather(_NPAD, embedding.shape)
    edge_gather = _make_sc_gather(_EPAD, (_NPAD, _AF // 2))
    crys_gather = _make_sc_gather(_BATCH * _L, (_NPAD, _HID))

    af = embed_gather(embedding, an_pad)
    p, af_bf = _matmul_bias(af, convW[0, :_AF, :], convB[0].reshape(1, -1))

    for i in range(_NCONV):
        wn = convW[i, _AF:2 * _AF, :]
        wnf = convW[i, 2 * _AF:, :]
        gat = edge_gather(af_bf, flat_idx).reshape(_EPAD // 2, _AF)
        s1, s2 = _edge_stats(gat, xfe, xfo, p, wn.astype(jnp.bfloat16), wnf)
        mu = s1[0] / _EDGES
        var = s2[0] / _EDGES - mu * mu
        a1 = bn1_g[i] * lax.rsqrt(var + 1e-5)
        c1 = bn1_b[i] - mu * a1
        ns, t1, t2 = _edge_apply(gat, xfe, xfo, p,
                                 (wn * a1[None, :]).astype(jnp.bfloat16),
                                 wnf * a1[None, :], jnp.stack([a1, c1]))
        mu2 = t1[0] / _N
        var2 = t2[0] / _N - mu2 * mu2
        a2 = bn2_g[i] * lax.rsqrt(var2 + 1e-5)
        c2 = bn2_b[i] - mu2 * a2
        if i < _NCONV - 1:
            wnxt, bnxt = convW[i + 1, :_AF, :], convB[i + 1].reshape(1, -1)
        else:
            wnxt, bnxt = fcW, fcb.reshape(1, -1)
        af, p, af_bf = _update(af, ns, jnp.stack([a2, c2]), wnxt, bnxt)

    crys = crystal_atom_idx.reshape(-1).astype(jnp.int32)
    gat = crys_gather(p, crys)
    new_atom_fea = jnp.concatenate(
        [gat.reshape(_BATCH, _L, _HID),
         jnp.zeros((_BATCH, _MAXG - _L, _HID), f32)], axis=1)
    mask = jnp.concatenate(
        [jnp.ones((_BATCH, _L), f32),
         jnp.zeros((_BATCH, _MAXG - _L), f32)], axis=1)
    return (new_atom_fea, mask)


# speedup vs baseline: 1.2352x; 1.2352x over previous
import functools

import jax
import jax.numpy as jnp
from jax import lax
from jax.experimental import pallas as pl
from jax.experimental.pallas import tpu as pltpu
from jax.experimental.pallas import tpu_sc as plsc

_AF = 128
_NF = 16
_M = 32
_N = 10000
_NPAD = 10240
_EPAD = _NPAD * _M
_EDGES = _N * _M
_HID = 256
_NCONV = 3
_BATCH = 32
_L = 256
_MAXG = 512
_BLK = 128
_GRID = _NPAD // _BLK
_BLKR = 512
_GRIDR = _NPAD // _BLKR

_NC, _NS = 2, 16
_NW = _NC * _NS


def _make_sc_gather(n_rows, table_shape, dtype=jnp.float32):
    b_per_w = n_rows // _NW
    chunk = 128
    while b_per_w % chunk != 0:
        chunk -= 8
    n_chunks = b_per_w // chunk
    assert n_chunks % 2 == 0
    d = table_shape[1]
    mesh = plsc.VectorSubcoreMesh(core_axis_name="c", subcore_axis_name="s")

    @functools.partial(
        pl.kernel,
        mesh=mesh,
        compiler_params=pltpu.CompilerParams(use_tc_tiling_on_sc=False),
        out_type=jax.ShapeDtypeStruct((n_rows, d), dtype),
        scratch_types=[
            pltpu.VMEM((b_per_w,), jnp.int32),
            pltpu.VMEM((chunk, d), dtype),
            pltpu.VMEM((chunk, d), dtype),
            pltpu.SemaphoreType.DMA,
            pltpu.SemaphoreType.DMA,
            pltpu.SemaphoreType.DMA,
            pltpu.SemaphoreType.DMA,
        ],
    )
    def gather_k(table_hbm, idx_hbm, out_hbm, idx_all, rows0, rows1,
                 sg0, sg1, so0, so1):
        wid = lax.axis_index("s") * _NC + lax.axis_index("c")
        base_w = wid * b_per_w
        pltpu.sync_copy(idx_hbm.at[pl.ds(base_w, b_per_w)], idx_all)
        rows = (rows0, rows1)
        sg = (sg0, sg1)
        so = (so0, so1)

        def gather_cp(c, b):
            return pltpu.make_async_copy(
                table_hbm.at[idx_all.at[pl.ds(c * chunk, chunk)]],
                rows[b], sg[b])

        gather_cp(0, 0).start()
        gather_cp(1, 1).start()

        def slot(c, b):
            gather_cp(c, b).wait()
            pltpu.async_copy(
                rows[b], out_hbm.at[pl.ds(base_w + c * chunk, chunk)],
                so[b]).wait()

            @pl.when(c + 2 < n_chunks)
            def _():
                gather_cp(c + 2, b).start()

        def body(t, carry):
            slot(t * 2, 0)
            slot(t * 2 + 1, 1)
            return carry

        lax.fori_loop(0, n_chunks // 2, body, 0)

    return gather_k


def _softplus(x):
    return jnp.maximum(x, 0.0) + jnp.log1p(jnp.exp(-jnp.abs(x)))


def _pack_bf16(x):
    u = lax.bitcast_convert_type(x, jnp.uint32)
    u = u + jnp.uint32(0x7FFF) + ((u >> 16) & jnp.uint32(1))
    lo = u[:, :_AF // 2] >> 16
    hi = u[:, _AF // 2:] & jnp.uint32(0xFFFF0000)
    return lax.bitcast_convert_type(lo | hi, jnp.float32)


def _unpack_matmul2(gp, wn):
    u = lax.bitcast_convert_type(gp, jnp.uint32)
    lo = lax.bitcast_convert_type(u << 16, jnp.float32).astype(jnp.bfloat16)
    hi = lax.bitcast_convert_type(u & jnp.uint32(0xFFFF0000),
                                  jnp.float32).astype(jnp.bfloat16)
    h = _AF // 2
    z_e = (jnp.dot(lo[:, :h], wn[:h], preferred_element_type=jnp.float32)
           + jnp.dot(hi[:, :h], wn[h:], preferred_element_type=jnp.float32))
    z_o = (jnp.dot(lo[:, h:], wn[:h], preferred_element_type=jnp.float32)
           + jnp.dot(hi[:, h:], wn[h:], preferred_element_type=jnp.float32))
    return z_e, z_o


def _rowmask(i, blk):
    aid = i * blk + lax.broadcasted_iota(jnp.int32, (blk, 1), 0)
    return (aid < _N).astype(jnp.float32)


def _k0_body(x_ref, w_ref, b_ref, o_ref, xb_ref):
    x = x_ref[...] * _rowmask(pl.program_id(0), _BLKR)
    o_ref[...] = (
        jnp.dot(x, w_ref[...], preferred_element_type=jnp.float32)
        + b_ref[...]
    ) * _rowmask(pl.program_id(0), _BLKR)
    xb_ref[...] = _pack_bf16(x)


def _matmul_bias(x, w, b):
    kdim, odim = w.shape
    return pl.pallas_call(
        _k0_body,
        grid=(_GRIDR,),
        in_specs=[
            pl.BlockSpec((_BLKR, kdim), lambda i: (i, 0)),
            pl.BlockSpec((kdim, odim), lambda i: (0, 0)),
            pl.BlockSpec((1, odim), lambda i: (0, 0)),
        ],
        out_specs=[
            pl.BlockSpec((_BLKR, odim), lambda i: (i, 0)),
            pl.BlockSpec((_BLKR, kdim // 2), lambda i: (i, 0)),
        ],
        out_shape=[
            jax.ShapeDtypeStruct((_NPAD, odim), jnp.float32),
            jax.ShapeDtypeStruct((_NPAD, kdim // 2), jnp.float32),
        ],
    )(x, w, b)


def _stats_body(g_ref, xfe_ref, xfo_ref, pe_ref, po_ref, wn_ref, wnf_ref,
                s1_ref, s2_ref):
    i = pl.program_id(0)

    @pl.when(i == 0)
    def _():
        s1_ref[...] = jnp.zeros_like(s1_ref)
        s2_ref[...] = jnp.zeros_like(s2_ref)

    z_e, z_o = _unpack_matmul2(g_ref[...], wn_ref[...])
    e_e = lax.dot_general(xfe_ref[...], wnf_ref[...], (((0,), (0,)), ((), ())),
                          preferred_element_type=jnp.float32)
    e_o = lax.dot_general(xfo_ref[...], wnf_ref[...], (((0,), (0,)), ((), ())),
                          preferred_element_type=jnp.float32)
    ge = (z_e + e_e).reshape(_BLK // 2, _M, 2 * _AF) + pe_ref[...][:, None, :]
    go = (z_o + e_o).reshape(_BLK // 2, _M, 2 * _AF) + po_ref[...][:, None, :]
    s1_ref[...] += (jnp.sum(ge, axis=(0, 1))
                    + jnp.sum(go, axis=(0, 1)))[None, :]
    s2_ref[...] += (jnp.sum(ge * ge, axis=(0, 1))
                    + jnp.sum(go * go, axis=(0, 1)))[None, :]


def _edge_stats(gat, xft, p, wn, wnf):
    return pl.pallas_call(
        _stats_body,
        grid=(_GRID,),
        in_specs=[
            pl.BlockSpec((_BLK * _M // 2, _AF), lambda i: (i, 0)),
            pl.BlockSpec((_NF, _BLK * _M // 2), lambda i: (0, i)),
            pl.BlockSpec((_NF, _BLK * _M // 2), lambda i: (0, i + _GRID)),
            pl.BlockSpec((_BLK // 2, 2 * _AF), lambda i: (i, 0)),
            pl.BlockSpec((_BLK // 2, 2 * _AF), lambda i: (i + _GRID, 0)),
            pl.BlockSpec((_AF, 2 * _AF), lambda i: (0, 0)),
            pl.BlockSpec((_NF, 2 * _AF), lambda i: (0, 0)),
        ],
        out_specs=[
            pl.BlockSpec((1, 2 * _AF), lambda i: (0, 0)),
            pl.BlockSpec((1, 2 * _AF), lambda i: (0, 0)),
        ],
        out_shape=[
            jax.ShapeDtypeStruct((1, 2 * _AF), jnp.float32),
            jax.ShapeDtypeStruct((1, 2 * _AF), jnp.float32),
        ],
    )(gat, xft, xft, p, p, wn, wnf)


def _apply_body(g_ref, xfe_ref, xfo_ref, pe_ref, po_ref, wn_ref, wnf_ref,
                ac_ref, nse_ref, nso_ref, t1_ref, t2_ref):
    i = pl.program_id(0)

    @pl.when(i == 0)
    def _():
        t1_ref[...] = jnp.zeros_like(t1_ref)
        t2_ref[...] = jnp.zeros_like(t2_ref)

    z_e, z_o = _unpack_matmul2(g_ref[...], wn_ref[...])
    e_e = lax.dot_general(xfe_ref[...], wnf_ref[...], (((0,), (0,)), ((), ())),
                          preferred_element_type=jnp.float32)
    e_o = lax.dot_general(xfo_ref[...], wnf_ref[...], (((0,), (0,)), ((), ())),
                          preferred_element_type=jnp.float32)
    a = ac_ref[0:1, :]
    c = ac_ref[1:2, :]
    pbe = (pe_ref[...] * a + c)[:, None, :]
    pbo = (po_ref[...] * a + c)[:, None, :]
    ghe = (z_e + e_e).reshape(_BLK // 2, _M, 2 * _AF) + pbe
    gho = (z_o + e_o).reshape(_BLK // 2, _M, 2 * _AF) + pbo
    red_e = jnp.sum(jax.nn.sigmoid(ghe[:, :, :_AF])
                    * _softplus(ghe[:, :, _AF:]), axis=1)
    red_o = jnp.sum(jax.nn.sigmoid(gho[:, :, :_AF])
                    * _softplus(gho[:, :, _AF:]), axis=1)
    nse_ref[...] = red_e
    nso_ref[...] = red_o
    aid = (_NPAD // 2 + i * (_BLK // 2)
           + lax.broadcasted_iota(jnp.int32, (_BLK // 2, 1), 0))
    rm = red_o * (aid < _N).astype(jnp.float32)
    t1_ref[...] += (jnp.sum(red_e, axis=0) + jnp.sum(rm, axis=0))[None, :]
    t2_ref[...] += (jnp.sum(red_e * red_e, axis=0)
                    + jnp.sum(rm * red_o, axis=0))[None, :]


def _edge_apply(gat, xft, p, wn, wnf, ac):
    return pl.pallas_call(
        _apply_body,
        grid=(_GRID,),
        in_specs=[
            pl.BlockSpec((_BLK * _M // 2, _AF), lambda i: (i, 0)),
            pl.BlockSpec((_NF, _BLK * _M // 2), lambda i: (0, i)),
            pl.BlockSpec((_NF, _BLK * _M // 2), lambda i: (0, i + _GRID)),
            pl.BlockSpec((_BLK // 2, 2 * _AF), lambda i: (i, 0)),
            pl.BlockSpec((_BLK // 2, 2 * _AF), lambda i: (i + _GRID, 0)),
            pl.BlockSpec((_AF, 2 * _AF), lambda i: (0, 0)),
            pl.BlockSpec((_NF, 2 * _AF), lambda i: (0, 0)),
            pl.BlockSpec((2, 2 * _AF), lambda i: (0, 0)),
        ],
        out_specs=[
            pl.BlockSpec((_BLK // 2, _AF), lambda i: (i, 0)),
            pl.BlockSpec((_BLK // 2, _AF), lambda i: (i, 0)),
            pl.BlockSpec((1, _AF), lambda i: (0, 0)),
            pl.BlockSpec((1, _AF), lambda i: (0, 0)),
        ],
        out_shape=[
            jax.ShapeDtypeStruct((_NPAD // 2, _AF), jnp.float32),
            jax.ShapeDtypeStruct((_NPAD // 2, _AF), jnp.float32),
            jax.ShapeDtypeStruct((1, _AF), jnp.float32),
            jax.ShapeDtypeStruct((1, _AF), jnp.float32),
        ],
    )(gat, xft, xft, p, p, wn, wnf, ac)


def _update_body(af_ref, ns_ref, ac2_ref, w_ref, b_ref,
                 afn_ref, pn_ref, afb_ref):
    a2 = ac2_ref[0:1, :]
    c2 = ac2_ref[1:2, :]
    m = _rowmask(pl.program_id(0), _BLKR)
    afn = _softplus(af_ref[...] + ns_ref[...] * a2 + c2) * m
    afn_ref[...] = afn
    afb_ref[...] = _pack_bf16(afn)
    pn_ref[...] = (
        jnp.dot(afn, w_ref[...], preferred_element_type=jnp.float32)
        + b_ref[...]
    ) * m


def _update(af, ns, ac2, w, b):
    odim = w.shape[1]
    return pl.pallas_call(
        _update_body,
        grid=(_GRIDR,),
        in_specs=[
            pl.BlockSpec((_BLKR, _AF), lambda i: (i, 0)),
            pl.BlockSpec((_BLKR, _AF), lambda i: (i, 0)),
            pl.BlockSpec((2, _AF), lambda i: (0, 0)),
            pl.BlockSpec((_AF, odim), lambda i: (0, 0)),
            pl.BlockSpec((1, odim), lambda i: (0, 0)),
        ],
        out_specs=[
            pl.BlockSpec((_BLKR, _AF), lambda i: (i, 0)),
            pl.BlockSpec((_BLKR, odim), lambda i: (i, 0)),
            pl.BlockSpec((_BLKR, _AF // 2), lambda i: (i, 0)),
        ],
        out_shape=[
            jax.ShapeDtypeStruct((_NPAD, _AF), jnp.float32),
            jax.ShapeDtypeStruct((_NPAD, odim), jnp.float32),
            jax.ShapeDtypeStruct((_NPAD, _AF // 2), jnp.float32),
        ],
    )(af, ns, ac2, w, b)


def kernel(atom_num, nbr_idx, nbr_fea, crystal_atom_idx, embedding,
           convW, convB, bn1_g, bn1_b, bn2_g, bn2_b, fcW, fcb):
    f32 = jnp.float32
    an_pad = jnp.concatenate(
        [atom_num.astype(jnp.int32), jnp.zeros((_NPAD - _N,), jnp.int32)])
    flat0 = jnp.concatenate(
        [nbr_idx.reshape(-1).astype(jnp.int32),
         jnp.full((_EPAD - _EDGES,), _NPAD - 1, jnp.int32)])
    flat_idx = jnp.stack(
        [flat0[:_EPAD // 2], flat0[_EPAD // 2:]], axis=1).reshape(-1)
    xft = jnp.concatenate(
        [nbr_fea.reshape(_EDGES, _NF).T,
         jnp.zeros((_NF, _EPAD - _EDGES), f32)], axis=1)

    embed_gather = _make_sc_gather(_NPAD, embedding.shape)
    edge_gather = _make_sc_gather(_EPAD, (_NPAD, _AF // 2))
    crys_gather = _make_sc_gather(_BATCH * _L, (_NPAD, _HID))

    af = embed_gather(embedding, an_pad)
    p, af_bf = _matmul_bias(af, convW[0, :_AF, :], convB[0].reshape(1, -1))

    for i in range(_NCONV):
        wn = convW[i, _AF:2 * _AF, :]
        wnf = convW[i, 2 * _AF:, :]
        gat = edge_gather(af_bf, flat_idx).reshape(_EPAD // 2, _AF)
        s1, s2 = _edge_stats(gat, xft, p, wn.astype(jnp.bfloat16), wnf)
        mu = s1[0] / _EDGES
        var = s2[0] / _EDGES - mu * mu
        a1 = bn1_g[i] * lax.rsqrt(var + 1e-5)
        c1 = bn1_b[i] - mu * a1
        ns_e, ns_o, t1, t2 = _edge_apply(
            gat, xft, p, (wn * a1[None, :]).astype(jnp.bfloat16),
            wnf * a1[None, :], jnp.stack([a1, c1]))
        ns = jnp.concatenate([ns_e, ns_o], axis=0)
        mu2 = t1[0] / _N
        var2 = t2[0] / _N - mu2 * mu2
        a2 = bn2_g[i] * lax.rsqrt(var2 + 1e-5)
        c2 = bn2_b[i] - mu2 * a2
        if i < _NCONV - 1:
            wnxt, bnxt = convW[i + 1, :_AF, :], convB[i + 1].reshape(1, -1)
        else:
            wnxt, bnxt = fcW, fcb.reshape(1, -1)
        af, p, af_bf = _update(af, ns, jnp.stack([a2, c2]), wnxt, bnxt)

    crys = crystal_atom_idx.reshape(-1).astype(jnp.int32)
    gat = crys_gather(p, crys)
    new_atom_fea = jnp.concatenate(
        [gat.reshape(_BATCH, _L, _HID),
         jnp.zeros((_BATCH, _MAXG - _L, _HID), f32)], axis=1)
    mask = jnp.concatenate(
        [jnp.ones((_BATCH, _L), f32),
         jnp.zeros((_BATCH, _MAXG - _L), f32)], axis=1)
    return (new_atom_fea, mask)
